# baseline (speedup 1.0000x reference)
import functools

import jax
import jax.numpy as jnp
from jax import lax
from jax.experimental import pallas as pl
from jax.experimental.pallas import tpu as pltpu
from jax.experimental.pallas import tpu_sc as plsc

_V = 1000000
_DIM = 64
_B = 4096
_L = 200
_N = _B * _L
_NC = 2
_NS = 16
_NW = _NC * _NS

_mesh = plsc.VectorSubcoreMesh(core_axis_name="c", subcore_axis_name="s")

_NFULL = _V // 128
_K1_ITERS = _NFULL // _NW + 1


def _k1_body(tab_t, out, buf0, buf1, tbuf, ow0, ow1, out_v2, sem0, sem1, os0, os1):
    wid = lax.axis_index("s") * _NC + lax.axis_index("c")
    iota = lax.iota(jnp.int32, 16)

    bufs = (buf0, buf1)
    sems = (sem0, sem1)
    out_vs = (ow0, ow1)
    osems = (os0, os1)

    def gstart(c, b):
        pltpu.async_copy(tab_t.at[:, pl.ds(128 * c, 128)], bufs[b], sems[b])

    def gwait(c, b):
        pltpu.make_async_copy(
            tab_t.at[:, pl.ds(128 * c, 128)], bufs[b], sems[b]
        ).wait()

    vds = [16 * jc + iota for jc in range(8)]
    v64s = [64 * v for v in vds]

    def transpose_block(buf, dst):
        @plsc.parallel_loop(0, 64, unroll=8)
        def fbody(f0):
            fd = lax.bitwise_and(f0 + iota, 63)
            for jc in range(8):
                vec = plsc.load_gather(buf, [fd, vds[jc]])
                plsc.store_scatter(dst, [v64s[jc] + fd], vec)

    gstart(wid, 0)

    def step(g, b):
        c = g * _NW + wid
        gwait(c, b)

        nxt = (g + 1) * _NW + wid

        @pl.when(nxt < _NFULL)
        def _():
            gstart(nxt, 1 - b)

        @pl.when(g >= 2)
        def _():
            pltpu.make_async_copy(
                out_vs[b], out.at[pl.ds(8192 * wid, 8192)], osems[b]
            ).wait()

        transpose_block(bufs[b], out_vs[b])
        pltpu.async_copy(out_vs[b], out.at[pl.ds(8192 * c, 8192)], osems[b])

    def outer(i, carry):
        for b in range(2):
            g = 2 * i + b

            @pl.when(g * _NW + wid < _NFULL)
            def _():
                step(g, b)

        return carry

    lax.fori_loop(0, (_K1_ITERS + 1) // 2, outer, 0)

    for b in range(2):
        pltpu.make_async_copy(
            out_vs[b], out.at[pl.ds(8192 * wid, 8192)], osems[b]
        ).wait()

    @pl.when(wid == _NW - 1)
    def _():
        pltpu.sync_copy(tab_t.at[:, pl.ds(128 * _NFULL, 64)], tbuf)

        @plsc.parallel_loop(0, 64, unroll=8)
        def fbody(f0):
            fd = lax.bitwise_and(f0 + iota, 63)
            for jc in range(4):
                vec = plsc.load_gather(tbuf, [fd, vds[jc]])
                plsc.store_scatter(out_v2, [v64s[jc] + fd], vec)

        pltpu.sync_copy(out_v2, out.at[pl.ds(8192 * _NFULL, 4096)])


_k1 = functools.partial(
    pl.kernel,
    mesh=_mesh,
    out_type=jax.ShapeDtypeStruct((_V * _DIM,), jnp.float32),
    scratch_types=[
        pltpu.VMEM((_DIM, 128), jnp.float32),
        pltpu.VMEM((_DIM, 128), jnp.float32),
        pltpu.VMEM((_DIM, 64), jnp.float32),
        pltpu.VMEM((8192,), jnp.float32),
        pltpu.VMEM((8192,), jnp.float32),
        pltpu.VMEM((4096,), jnp.float32),
        pltpu.SemaphoreType.DMA,
        pltpu.SemaphoreType.DMA,
        pltpu.SemaphoreType.DMA,
        pltpu.SemaphoreType.DMA,
    ],
    compiler_params=pltpu.CompilerParams(use_tc_tiling_on_sc=True, needs_layout_passes=False),
)(_k1_body)


_CHUNK = 512


def _k2_body(table, idx_hbm, out, idx_v, rows0, rows1, sem0, sem1):
    wid = lax.axis_index("s") * _NC + lax.axis_index("c")
    b_per_w = _N // _NW
    base = wid * b_per_w
    nchunks = b_per_w // _CHUNK

    pltpu.sync_copy(idx_hbm.at[pl.ds(base, b_per_w)], idx_v)

    rows = (rows0, rows1)
    sems = (sem0, sem1)

    def gstart(g, b):
        pltpu.async_copy(
            table.at[idx_v.at[pl.ds(g * _CHUNK, _CHUNK)]], rows[b], sems[b]
        )

    def gwait(g, b):
        pltpu.make_async_copy(
            table.at[idx_v.at[pl.ds(g * _CHUNK, _CHUNK)]], rows[b], sems[b]
        ).wait()

    gstart(0, 0)

    def outer(i, carry):
        for b in range(2):
            g = 2 * i + b
            gwait(g, b)

            @pl.when(g + 1 < nchunks)
            def _():
                gstart(g + 1, (b + 1) % 2)

            pltpu.sync_copy(rows[b], out.at[pl.ds(base + g * _CHUNK, _CHUNK)])
        return carry

    lax.fori_loop(0, nchunks // 2, outer, 0)


_k2 = functools.partial(
    pl.kernel,
    mesh=_mesh,
    out_type=jax.ShapeDtypeStruct((_N, _DIM), jnp.float32),
    scratch_types=[
        pltpu.VMEM((_N // _NW,), jnp.int32),
        pltpu.VMEM((_CHUNK, _DIM), jnp.float32),
        pltpu.VMEM((_CHUNK, _DIM), jnp.float32),
        pltpu.SemaphoreType.DMA,
        pltpu.SemaphoreType.DMA,
    ],
    compiler_params=pltpu.CompilerParams(use_tc_tiling_on_sc=False),
)(_k2_body)


def _k23_body(table, idx_hbm, out5, idx_all, idx_lm, g0, g1, ow0, ow1, sem0, sem1, os0, os1):
    wid = lax.axis_index("s") * _NC + lax.axis_index("c")
    iota = lax.iota(jnp.int32, 16)
    gbufs = (g0, g1)
    sems = (sem0, sem1)
    out_vs = (ow0, ow1)
    osems = (os0, os1)
    n_per_w = _N // _NW

    pltpu.sync_copy(idx_hbm.at[pl.ds(wid * n_per_w, n_per_w)], idx_all)

    jvs = [16 * jc + iota for jc in range(8)]
    s200j = [200 * jv for jv in jvs]

    def reorder(l0, carry):
        ld = lax.rem(l0 + iota, 200)
        ld128 = 128 * ld
        for jc in range(8):
            vec = plsc.load_gather(idx_all, [s200j[jc] + ld])
            plsc.store_scatter(idx_lm, [ld128 + jvs[jc]], vec)
        return carry

    lax.fori_loop(0, _L, reorder, 0)

    def gstart(l, p):
        pltpu.async_copy(
            table.at[idx_lm.at[pl.ds(128 * l, 128)]], gbufs[p], sems[p]
        )

    def gwait(l, p):
        pltpu.make_async_copy(
            table.at[idx_lm.at[pl.ds(128 * l, 128)]], gbufs[p], sems[p]
        ).wait()

    gstart(0, 0)

    def step(l, p):
        gwait(l, p)

        @pl.when(l + 1 < _L)
        def _():
            gstart(l + 1, 1 - p)

        @pl.when(l >= 2)
        def _():
            pltpu.make_async_copy(
                out_vs[p], out5.at[0, :, wid, :, :], osems[p]
            ).wait()

        gbuf = gbufs[p]
        dst = out_vs[p]

        @plsc.parallel_loop(0, 64, unroll=8)
        def fbody(f0):
            fd = lax.bitwise_and(f0 + iota, 63)
            fhi = lax.shift_right_logical(fd, 3)
            flo = lax.bitwise_and(fd, 7)
            for jc in range(8):
                vec = plsc.load_gather(gbuf, [jvs[jc], fd])
                plsc.store_scatter(dst, [fhi, flo, jvs[jc]], vec)

        pltpu.async_copy(dst, out5.at[l, :, wid, :, :], osems[p])

    def outer(i, carry):
        for p in range(2):
            step(2 * i + p, p)
        return carry

    lax.fori_loop(0, _L // 2, outer, 0)
    for p in range(2):
        pltpu.make_async_copy(
            out_vs[p], out5.at[0, :, wid, :, :], osems[p]
        ).wait()


_k23 = functools.partial(
    pl.kernel,
    mesh=_mesh,
    out_type=jax.ShapeDtypeStruct((_L, 8, 32, 8, 128), jnp.float32),
    scratch_types=[
        pltpu.VMEM((_N // _NW,), jnp.int32),
        pltpu.VMEM((_N // _NW,), jnp.int32),
        pltpu.VMEM((128, _DIM), jnp.float32),
        pltpu.VMEM((128, _DIM), jnp.float32),
        pltpu.VMEM((8, 8, 128), jnp.float32),
        pltpu.VMEM((8, 8, 128), jnp.float32),
        pltpu.SemaphoreType.DMA,
        pltpu.SemaphoreType.DMA,
        pltpu.SemaphoreType.DMA,
        pltpu.SemaphoreType.DMA,
    ],
    compiler_params=pltpu.CompilerParams(
        use_tc_tiling_on_sc=False, needs_layout_passes=False
    ),
)(_k23_body)


def _k3_body(in2, out3, idx_v, g0, g1, ow0, ow1, sem0, sem1, os0, os1):
    wid = lax.axis_index("s") * _NC + lax.axis_index("c")
    iota = lax.iota(jnp.int32, 16)
    out_vs = (ow0, ow1)
    osems = (os0, os1)
    jvs = [16 * jc + iota for jc in range(8)]

    def set_idx(l):
        for jc in range(8):
            j = 16 * jc + iota
            idx_v[pl.ds(16 * jc, 16)] = 12800 * wid + 100 * j + lax.shift_right_logical(l, 1)

    gbufs = (g0, g1)
    sems = (sem0, sem1)

    def gstart(b):
        pltpu.async_copy(in2.at[idx_v], gbufs[b], sems[b])

    def gwait(b):
        pltpu.make_async_copy(in2.at[idx_v], gbufs[b], sems[b]).wait()

    set_idx(jnp.int32(0))
    gstart(0)

    def step(l, p):
        gwait(p)

        @pl.when(l + 1 < _L)
        def _():
            set_idx(l + 1)
            gstart(1 - p)

        @pl.when(l >= 2)
        def _():
            pltpu.make_async_copy(
                out_vs[p], out3.at[0, :, pl.ds(128 * wid, 128)], osems[p]
            ).wait()

        gbuf = gbufs[p]
        dst = out_vs[p]

        @plsc.parallel_loop(0, 64, unroll=8)
        def fbody(f0):
            fd = lax.bitwise_and(f0 + iota, 63)
            for jc in range(8):
                vec = plsc.load_gather(gbuf, [jvs[jc], fd + 64 * p])
                plsc.store_scatter(dst, [fd, jvs[jc]], vec)

        pltpu.async_copy(dst, out3.at[l, :, pl.ds(128 * wid, 128)], osems[p])

    def outer(i, carry):
        for p in range(2):
            step(2 * i + p, p)
        return carry

    lax.fori_loop(0, _L // 2, outer, 0)
    for p in range(2):
        pltpu.make_async_copy(
            out_vs[p], out3.at[0, :, pl.ds(128 * wid, 128)], osems[p]
        ).wait()


_k3 = functools.partial(
    pl.kernel,
    mesh=_mesh,
    out_type=jax.ShapeDtypeStruct((_L, _DIM, _B), jnp.float32),
    scratch_types=[
        pltpu.VMEM((128,), jnp.int32),
        pltpu.VMEM((128, 128), jnp.float32),
        pltpu.VMEM((128, 128), jnp.float32),
        pltpu.VMEM((_DIM, 128), jnp.float32),
        pltpu.VMEM((_DIM, 128), jnp.float32),
        pltpu.SemaphoreType.DMA,
        pltpu.SemaphoreType.DMA,
        pltpu.SemaphoreType.DMA,
        pltpu.SemaphoreType.DMA,
    ],
    compiler_params=pltpu.CompilerParams(use_tc_tiling_on_sc=True, needs_layout_passes=False),
)(_k3_body)


def kernel(embedding, x):
    b, l = x.shape
    flat = x.reshape(b * l).astype(jnp.int32)
    tab = _k1(embedding.T)
    out5 = _k23(tab.reshape(_V, _DIM), flat)
    return out5.transpose(2, 4, 0, 1, 3).reshape(b, l, _DIM)


# speedup vs baseline: 1.0145x; 1.0145x over previous
import functools

import jax
import jax.numpy as jnp
from jax import lax
from jax.experimental import pallas as pl
from jax.experimental.pallas import tpu as pltpu
from jax.experimental.pallas import tpu_sc as plsc

_V = 1000000
_DIM = 64
_B = 4096
_L = 200
_N = _B * _L
_NC = 2
_NS = 16
_NW = _NC * _NS

_mesh = plsc.VectorSubcoreMesh(core_axis_name="c", subcore_axis_name="s")

_NFULL = _V // 128
_K1_ITERS = _NFULL // _NW + 1


def _k1_body(tab_t, out, buf0, buf1, tbuf, ow0, ow1, out_v2, sem0, sem1, os0, os1):
    wid = lax.axis_index("s") * _NC + lax.axis_index("c")
    iota = lax.iota(jnp.int32, 16)

    bufs = (buf0, buf1)
    sems = (sem0, sem1)
    out_vs = (ow0, ow1)
    osems = (os0, os1)

    def gstart(c, b):
        pltpu.async_copy(tab_t.at[:, pl.ds(128 * c, 128)], bufs[b], sems[b])

    def gwait(c, b):
        pltpu.make_async_copy(
            tab_t.at[:, pl.ds(128 * c, 128)], bufs[b], sems[b]
        ).wait()

    vds = [16 * jc + iota for jc in range(8)]
    v64s = [64 * v for v in vds]

    def transpose_block(buf, dst):
        @plsc.parallel_loop(0, 64, unroll=8)
        def fbody(f0):
            fd = lax.bitwise_and(f0 + iota, 63)
            for jc in range(8):
                vec = plsc.load_gather(buf, [fd, vds[jc]])
                plsc.store_scatter(dst, [v64s[jc] + fd], vec)

    gstart(wid, 0)

    def step(g, b):
        c = g * _NW + wid
        gwait(c, b)

        nxt = (g + 1) * _NW + wid

        @pl.when(nxt < _NFULL)
        def _():
            gstart(nxt, 1 - b)

        @pl.when(g >= 2)
        def _():
            pltpu.make_async_copy(
                out_vs[b], out.at[pl.ds(8192 * wid, 8192)], osems[b]
            ).wait()

        transpose_block(bufs[b], out_vs[b])
        pltpu.async_copy(out_vs[b], out.at[pl.ds(8192 * c, 8192)], osems[b])

    def outer(i, carry):
        for b in range(2):
            g = 2 * i + b

            @pl.when(g * _NW + wid < _NFULL)
            def _():
                step(g, b)

        return carry

    lax.fori_loop(0, (_K1_ITERS + 1) // 2, outer, 0)

    for b in range(2):
        pltpu.make_async_copy(
            out_vs[b], out.at[pl.ds(8192 * wid, 8192)], osems[b]
        ).wait()

    @pl.when(wid == _NW - 1)
    def _():
        pltpu.sync_copy(tab_t.at[:, pl.ds(128 * _NFULL, 64)], tbuf)

        @plsc.parallel_loop(0, 64, unroll=8)
        def fbody(f0):
            fd = lax.bitwise_and(f0 + iota, 63)
            for jc in range(4):
                vec = plsc.load_gather(tbuf, [fd, vds[jc]])
                plsc.store_scatter(out_v2, [v64s[jc] + fd], vec)

        pltpu.sync_copy(out_v2, out.at[pl.ds(8192 * _NFULL, 4096)])


_k1 = functools.partial(
    pl.kernel,
    mesh=_mesh,
    out_type=jax.ShapeDtypeStruct((_V * _DIM,), jnp.float32),
    scratch_types=[
        pltpu.VMEM((_DIM, 128), jnp.float32),
        pltpu.VMEM((_DIM, 128), jnp.float32),
        pltpu.VMEM((_DIM, 64), jnp.float32),
        pltpu.VMEM((8192,), jnp.float32),
        pltpu.VMEM((8192,), jnp.float32),
        pltpu.VMEM((4096,), jnp.float32),
        pltpu.SemaphoreType.DMA,
        pltpu.SemaphoreType.DMA,
        pltpu.SemaphoreType.DMA,
        pltpu.SemaphoreType.DMA,
    ],
    compiler_params=pltpu.CompilerParams(use_tc_tiling_on_sc=True, needs_layout_passes=False),
)(_k1_body)


_CHUNK = 512


def _k2_body(table, idx_hbm, out, idx_v, rows0, rows1, sem0, sem1):
    wid = lax.axis_index("s") * _NC + lax.axis_index("c")
    b_per_w = _N // _NW
    base = wid * b_per_w
    nchunks = b_per_w // _CHUNK

    pltpu.sync_copy(idx_hbm.at[pl.ds(base, b_per_w)], idx_v)

    rows = (rows0, rows1)
    sems = (sem0, sem1)

    def gstart(g, b):
        pltpu.async_copy(
            table.at[idx_v.at[pl.ds(g * _CHUNK, _CHUNK)]], rows[b], sems[b]
        )

    def gwait(g, b):
        pltpu.make_async_copy(
            table.at[idx_v.at[pl.ds(g * _CHUNK, _CHUNK)]], rows[b], sems[b]
        ).wait()

    gstart(0, 0)

    def outer(i, carry):
        for b in range(2):
            g = 2 * i + b
            gwait(g, b)

            @pl.when(g + 1 < nchunks)
            def _():
                gstart(g + 1, (b + 1) % 2)

            pltpu.sync_copy(rows[b], out.at[pl.ds(base + g * _CHUNK, _CHUNK)])
        return carry

    lax.fori_loop(0, nchunks // 2, outer, 0)


_k2 = functools.partial(
    pl.kernel,
    mesh=_mesh,
    out_type=jax.ShapeDtypeStruct((_N, _DIM), jnp.float32),
    scratch_types=[
        pltpu.VMEM((_N // _NW,), jnp.int32),
        pltpu.VMEM((_CHUNK, _DIM), jnp.float32),
        pltpu.VMEM((_CHUNK, _DIM), jnp.float32),
        pltpu.SemaphoreType.DMA,
        pltpu.SemaphoreType.DMA,
    ],
    compiler_params=pltpu.CompilerParams(use_tc_tiling_on_sc=False),
)(_k2_body)


def _k23_body(table, idx_t_hbm, out5, idx_lm, g0, g1, ow0, ow1, sem0, sem1, os0, os1):
    wid = lax.axis_index("s") * _NC + lax.axis_index("c")
    iota = lax.iota(jnp.int32, 16)
    gbufs = (g0, g1)
    sems = (sem0, sem1)
    out_vs = (ow0, ow1)
    osems = (os0, os1)

    pltpu.sync_copy(idx_t_hbm.at[:, pl.ds(128 * wid, 128)], idx_lm)

    jvs = [16 * jc + iota for jc in range(8)]

    def gstart(l, p):
        pltpu.async_copy(table.at[idx_lm.at[l]], gbufs[p], sems[p])

    def gwait(l, p):
        pltpu.make_async_copy(table.at[idx_lm.at[l]], gbufs[p], sems[p]).wait()

    gstart(0, 0)

    def step(l, p):
        gwait(l, p)

        @pl.when(l + 1 < _L)
        def _():
            gstart(l + 1, 1 - p)

        @pl.when(l >= 2)
        def _():
            pltpu.make_async_copy(
                out_vs[p], out5.at[0, :, wid, :, :], osems[p]
            ).wait()

        gbuf = gbufs[p]
        dst = out_vs[p]

        @plsc.parallel_loop(0, 64, unroll=8)
        def fbody(f0):
            fd = lax.bitwise_and(f0 + iota, 63)
            fhi = lax.shift_right_logical(fd, 3)
            flo = lax.bitwise_and(fd, 7)
            for jc in range(8):
                vec = plsc.load_gather(gbuf, [jvs[jc], fd])
                plsc.store_scatter(dst, [fhi, flo, jvs[jc]], vec)

        pltpu.async_copy(dst, out5.at[l, :, wid, :, :], osems[p])

    def outer(i, carry):
        for p in range(2):
            step(2 * i + p, p)
        return carry

    lax.fori_loop(0, _L // 2, outer, 0)
    for p in range(2):
        pltpu.make_async_copy(
            out_vs[p], out5.at[0, :, wid, :, :], osems[p]
        ).wait()


_k23 = functools.partial(
    pl.kernel,
    mesh=_mesh,
    out_type=jax.ShapeDtypeStruct((_L, 8, 32, 8, 128), jnp.float32),
    scratch_types=[
        pltpu.VMEM((_L, 128), jnp.int32),
        pltpu.VMEM((128, _DIM), jnp.float32),
        pltpu.VMEM((128, _DIM), jnp.float32),
        pltpu.VMEM((8, 8, 128), jnp.float32),
        pltpu.VMEM((8, 8, 128), jnp.float32),
        pltpu.SemaphoreType.DMA,
        pltpu.SemaphoreType.DMA,
        pltpu.SemaphoreType.DMA,
        pltpu.SemaphoreType.DMA,
    ],
    compiler_params=pltpu.CompilerParams(
        use_tc_tiling_on_sc=False, needs_layout_passes=False
    ),
)(_k23_body)


def _k3_body(in2, out3, idx_v, g0, g1, ow0, ow1, sem0, sem1, os0, os1):
    wid = lax.axis_index("s") * _NC + lax.axis_index("c")
    iota = lax.iota(jnp.int32, 16)
    out_vs = (ow0, ow1)
    osems = (os0, os1)
    jvs = [16 * jc + iota for jc in range(8)]

    def set_idx(l):
        for jc in range(8):
            j = 16 * jc + iota
            idx_v[pl.ds(16 * jc, 16)] = 12800 * wid + 100 * j + lax.shift_right_logical(l, 1)

    gbufs = (g0, g1)
    sems = (sem0, sem1)

    def gstart(b):
        pltpu.async_copy(in2.at[idx_v], gbufs[b], sems[b])

    def gwait(b):
        pltpu.make_async_copy(in2.at[idx_v], gbufs[b], sems[b]).wait()

    set_idx(jnp.int32(0))
    gstart(0)

    def step(l, p):
        gwait(p)

        @pl.when(l + 1 < _L)
        def _():
            set_idx(l + 1)
            gstart(1 - p)

        @pl.when(l >= 2)
        def _():
            pltpu.make_async_copy(
                out_vs[p], out3.at[0, :, pl.ds(128 * wid, 128)], osems[p]
            ).wait()

        gbuf = gbufs[p]
        dst = out_vs[p]

        @plsc.parallel_loop(0, 64, unroll=8)
        def fbody(f0):
            fd = lax.bitwise_and(f0 + iota, 63)
            for jc in range(8):
                vec = plsc.load_gather(gbuf, [jvs[jc], fd + 64 * p])
                plsc.store_scatter(dst, [fd, jvs[jc]], vec)

        pltpu.async_copy(dst, out3.at[l, :, pl.ds(128 * wid, 128)], osems[p])

    def outer(i, carry):
        for p in range(2):
            step(2 * i + p, p)
        return carry

    lax.fori_loop(0, _L // 2, outer, 0)
    for p in range(2):
        pltpu.make_async_copy(
            out_vs[p], out3.at[0, :, pl.ds(128 * wid, 128)], osems[p]
        ).wait()


_k3 = functools.partial(
    pl.kernel,
    mesh=_mesh,
    out_type=jax.ShapeDtypeStruct((_L, _DIM, _B), jnp.float32),
    scratch_types=[
        pltpu.VMEM((128,), jnp.int32),
        pltpu.VMEM((128, 128), jnp.float32),
        pltpu.VMEM((128, 128), jnp.float32),
        pltpu.VMEM((_DIM, 128), jnp.float32),
        pltpu.VMEM((_DIM, 128), jnp.float32),
        pltpu.SemaphoreType.DMA,
        pltpu.SemaphoreType.DMA,
        pltpu.SemaphoreType.DMA,
        pltpu.SemaphoreType.DMA,
    ],
    compiler_params=pltpu.CompilerParams(use_tc_tiling_on_sc=True, needs_layout_passes=False),
)(_k3_body)


def kernel(embedding, x):
    b, l = x.shape
    idx_t = x.T.astype(jnp.int32)
    tab = _k1(embedding.T)
    out5 = _k23(tab.reshape(_V, _DIM), idx_t)
    return out5.transpose(2, 4, 0, 1, 3).reshape(b, l, _DIM)


# speedup vs baseline: 1.0150x; 1.0005x over previous
import functools

import jax
import jax.numpy as jnp
from jax import lax
from jax.experimental import pallas as pl
from jax.experimental.pallas import tpu as pltpu
from jax.experimental.pallas import tpu_sc as plsc

_V = 1000000
_DIM = 64
_B = 4096
_L = 200
_N = _B * _L
_NC = 2
_NS = 16
_NW = _NC * _NS

_mesh = plsc.VectorSubcoreMesh(core_axis_name="c", subcore_axis_name="s")

_NFULL = _V // 128
_K1_ITERS = _NFULL // _NW + 1


def _k1_body(tab_t, out, buf0, buf1, tbuf, ow0, ow1, out_v2, sem0, sem1, os0, os1):
    wid = lax.axis_index("s") * _NC + lax.axis_index("c")
    iota = lax.iota(jnp.int32, 16)

    bufs = (buf0, buf1)
    sems = (sem0, sem1)
    out_vs = (ow0, ow1)
    osems = (os0, os1)

    def gstart(c, b):
        pltpu.async_copy(tab_t.at[:, pl.ds(128 * c, 128)], bufs[b], sems[b])

    def gwait(c, b):
        pltpu.make_async_copy(
            tab_t.at[:, pl.ds(128 * c, 128)], bufs[b], sems[b]
        ).wait()

    vds = [16 * jc + iota for jc in range(8)]
    v64s = [64 * v for v in vds]

    def transpose_block(buf, dst):
        @plsc.parallel_loop(0, 64, unroll=8)
        def fbody(f0):
            fd = lax.bitwise_and(f0 + iota, 63)
            for jc in range(8):
                vec = plsc.load_gather(buf, [fd, vds[jc]])
                plsc.store_scatter(dst, [v64s[jc] + fd], vec)

    gstart(wid, 0)

    def step(g, b):
        c = g * _NW + wid
        gwait(c, b)

        nxt = (g + 1) * _NW + wid

        @pl.when(nxt < _NFULL)
        def _():
            gstart(nxt, 1 - b)

        @pl.when(g >= 2)
        def _():
            pltpu.make_async_copy(
                out_vs[b], out.at[pl.ds(8192 * wid, 8192)], osems[b]
            ).wait()

        transpose_block(bufs[b], out_vs[b])
        pltpu.async_copy(out_vs[b], out.at[pl.ds(8192 * c, 8192)], osems[b])

    def outer(i, carry):
        for b in range(2):
            g = 2 * i + b

            @pl.when(g * _NW + wid < _NFULL)
            def _():
                step(g, b)

        return carry

    lax.fori_loop(0, (_K1_ITERS + 1) // 2, outer, 0)

    for b in range(2):
        pltpu.make_async_copy(
            out_vs[b], out.at[pl.ds(8192 * wid, 8192)], osems[b]
        ).wait()

    @pl.when(wid == _NW - 1)
    def _():
        pltpu.sync_copy(tab_t.at[:, pl.ds(128 * _NFULL, 64)], tbuf)

        @plsc.parallel_loop(0, 64, unroll=8)
        def fbody(f0):
            fd = lax.bitwise_and(f0 + iota, 63)
            for jc in range(4):
                vec = plsc.load_gather(tbuf, [fd, vds[jc]])
                plsc.store_scatter(out_v2, [v64s[jc] + fd], vec)

        pltpu.sync_copy(out_v2, out.at[pl.ds(8192 * _NFULL, 4096)])


_k1 = functools.partial(
    pl.kernel,
    mesh=_mesh,
    out_type=jax.ShapeDtypeStruct((_V * _DIM,), jnp.float32),
    scratch_types=[
        pltpu.VMEM((_DIM, 128), jnp.float32),
        pltpu.VMEM((_DIM, 128), jnp.float32),
        pltpu.VMEM((_DIM, 64), jnp.float32),
        pltpu.VMEM((8192,), jnp.float32),
        pltpu.VMEM((8192,), jnp.float32),
        pltpu.VMEM((4096,), jnp.float32),
        pltpu.SemaphoreType.DMA,
        pltpu.SemaphoreType.DMA,
        pltpu.SemaphoreType.DMA,
        pltpu.SemaphoreType.DMA,
    ],
    compiler_params=pltpu.CompilerParams(use_tc_tiling_on_sc=True, needs_layout_passes=False),
)(_k1_body)


def _k23_body(table, idx_t_hbm, out5, idx_lm, g0, g1, ow0, ow1, sem0, sem1, os0, os1):
    wid = lax.axis_index("s") * _NC + lax.axis_index("c")
    iota = lax.iota(jnp.int32, 16)
    gbufs = (g0, g1)
    sems = (sem0, sem1)
    out_vs = (ow0, ow1)
    osems = (os0, os1)

    pltpu.sync_copy(idx_t_hbm.at[:, pl.ds(128 * wid, 128)], idx_lm)

    jvs = [16 * jc + iota for jc in range(8)]

    def gstart(l, p):
        pltpu.async_copy(table.at[idx_lm.at[l]], gbufs[p], sems[p])

    def gwait(l, p):
        pltpu.make_async_copy(table.at[idx_lm.at[l]], gbufs[p], sems[p]).wait()

    gstart(0, 0)

    def step(l, p):
        gwait(l, p)

        @pl.when(l + 1 < _L)
        def _():
            gstart(l + 1, 1 - p)

        @pl.when(l >= 2)
        def _():
            pltpu.make_async_copy(
                out_vs[p], out5.at[0, :, wid, :, :], osems[p]
            ).wait()

        gbuf = gbufs[p]
        dst = out_vs[p]

        @plsc.parallel_loop(0, 64, unroll=8)
        def fbody(f0):
            fd = lax.bitwise_and(f0 + iota, 63)
            fhi = lax.shift_right_logical(fd, 3)
            flo = lax.bitwise_and(fd, 7)
            for jc in range(8):
                vec = plsc.load_gather(gbuf, [jvs[jc], fd])
                plsc.store_scatter(dst, [fhi, flo, jvs[jc]], vec)

        pltpu.async_copy(dst, out5.at[l, :, wid, :, :], osems[p])

    def outer(i, carry):
        for p in range(2):
            step(2 * i + p, p)
        return carry

    lax.fori_loop(0, _L // 2, outer, 0)
    for p in range(2):
        pltpu.make_async_copy(
            out_vs[p], out5.at[0, :, wid, :, :], osems[p]
        ).wait()


_k23 = functools.partial(
    pl.kernel,
    mesh=_mesh,
    out_type=jax.ShapeDtypeStruct((_L, 8, 32, 8, 128), jnp.float32),
    scratch_types=[
        pltpu.VMEM((_L, 128), jnp.int32),
        pltpu.VMEM((128, _DIM), jnp.float32),
        pltpu.VMEM((128, _DIM), jnp.float32),
        pltpu.VMEM((8, 8, 128), jnp.float32),
        pltpu.VMEM((8, 8, 128), jnp.float32),
        pltpu.SemaphoreType.DMA,
        pltpu.SemaphoreType.DMA,
        pltpu.SemaphoreType.DMA,
        pltpu.SemaphoreType.DMA,
    ],
    compiler_params=pltpu.CompilerParams(
        use_tc_tiling_on_sc=False, needs_layout_passes=False
    ),
)(_k23_body)


def kernel(embedding, x):
    b, l = x.shape
    idx_t = x.T.astype(jnp.int32)
    tab = _k1(embedding.T)
    out5 = _k23(tab.reshape(_V, _DIM), idx_t)
    return out5.transpose(2, 4, 0, 1, 3).reshape(b, l, _DIM)
